# baseline (speedup 1.0000x reference)
import functools

import jax
import jax.numpy as jnp
from jax import lax
from jax.experimental import pallas as pl
from jax.experimental.pallas import tpu as pltpu
from jax.experimental.pallas import tpu_sc as plsc

EMB = 64
SCALE = 8.0
NC, NS, LANES = 2, 16, 16
NW = NC * NS
GATHER_W = 128
W = 256
TBLK = 2048
OBLK = 1024


@functools.cache
def _table_rows(V: int):
    grid = (V + TBLK - 1) // TBLK

    def body(t_ref, o_ref):
        xt = jnp.transpose(t_ref[...], (1, 0))
        o_ref[...] = jnp.concatenate([xt, xt], axis=1)

    return pl.pallas_call(
        body,
        grid=(grid,),
        in_specs=[pl.BlockSpec((EMB, TBLK), lambda i: (0, i))],
        out_specs=pl.BlockSpec((TBLK, 2 * EMB), lambda i: (i, 0)),
        out_shape=jax.ShapeDtypeStruct((V, 2 * EMB), jnp.float32),
    )


@functools.cache
def _to_entry_layout(S: int, BD: int):
    def body(r_ref, o_ref):
        x = r_ref[...]
        o_ref[...] = jnp.transpose(x[:, :EMB], (1, 0))[None]

    return pl.pallas_call(
        body,
        grid=(S, BD // OBLK),
        in_specs=[
            pl.BlockSpec(
                (OBLK, 2 * EMB), lambda s, b: (s * (BD // OBLK) + b, 0)
            )
        ],
        out_specs=pl.BlockSpec((1, EMB, OBLK), lambda s, b: (s, 0, b)),
        out_shape=jax.ShapeDtypeStruct((S, EMB, BD), jnp.float32),
    )


@functools.cache
def _emb_lookup(B: int, V: int):
    per_worker = B // W // NW
    mesh = plsc.VectorSubcoreMesh(core_axis_name="c", subcore_axis_name="s")

    @functools.partial(
        pl.kernel,
        mesh=mesh,
        compiler_params=pltpu.CompilerParams(use_tc_tiling_on_sc=True),
        out_type=jax.ShapeDtypeStruct((B, 2 * EMB), jnp.float32),
        scratch_types=[
            pltpu.VMEM((2, W), jnp.int32),
            pltpu.VMEM((2, W, 2 * EMB), jnp.float32),
            pltpu.SemaphoreType.DMA((2,)),
            pltpu.SemaphoreType.DMA((2,)),
        ],
    )
    def k(trows, idx_hbm, out_hbm, xv, buf, gsem, osem):
        wid = lax.axis_index("s") * NC + lax.axis_index("c")
        base_t = wid * per_worker

        def load_and_gather(t, p):
            off = (base_t + t) * W
            pltpu.sync_copy(idx_hbm.at[pl.ds(off, W)], xv.at[p])
            for g in range(W // GATHER_W):
                pltpu.async_copy(
                    trows.at[xv.at[p, pl.ds(g * GATHER_W, GATHER_W)]],
                    buf.at[p, pl.ds(g * GATHER_W, GATHER_W)],
                    gsem.at[p],
                )

        def drain_gather(p):
            pltpu.make_async_copy(
                trows.at[pl.ds(0, W)], buf.at[p], gsem.at[p]
            ).wait()

        def drain_store(p):
            pltpu.make_async_copy(
                buf.at[p], out_hbm.at[pl.ds(0, W)], osem.at[p]
            ).wait()

        load_and_gather(0, 0)

        @pl.loop(0, per_worker // 2)
        def _(h):
            for p in range(2):
                t = 2 * h + p
                tn = t + 1

                @pl.when(tn < per_worker)
                def _():
                    load_and_gather(tn, 1 - p)

                drain_gather(p)

                @pl.when(t >= 2)
                def _():
                    drain_store(p)

                @pl.loop(0, W)
                def _(r):
                    for c in range(0, EMB, LANES):
                        buf.at[p, r, pl.ds(c, LANES)][...] = (
                            buf.at[p, r, pl.ds(c, LANES)][...] * SCALE
                        )

                pltpu.async_copy(
                    buf.at[p],
                    out_hbm.at[pl.ds((base_t + t) * W, W)],
                    osem.at[p],
                )

        for p in range(2):
            drain_store(p)

    return k


def kernel(x, table):
    BD, S = x.shape
    V = table.shape[0]
    B = BD * S
    trows = _table_rows(V)(jnp.swapaxes(table, 0, 1))
    idx = jnp.swapaxes(x, 0, 1).reshape(-1).astype(jnp.int32)
    rows = _emb_lookup(B, V)(trows, idx)
    out = _to_entry_layout(S, BD)(rows)
    return jnp.transpose(out, (2, 0, 1))


# speedup vs baseline: 1.1262x; 1.1262x over previous
import functools

import jax
import jax.numpy as jnp
from jax import lax
from jax.experimental import pallas as pl
from jax.experimental.pallas import tpu as pltpu
from jax.experimental.pallas import tpu_sc as plsc

EMB = 64
SCALE = 8.0
NC, NS, LANES = 2, 16, 16
NW = NC * NS
GATHER_W = 128
CHUNK = 256
NBUF = 4


@functools.cache
def _emb_lookup(B: int):
    b_per_w = B // NW
    n_chunks = b_per_w // CHUNK
    mesh = plsc.VectorSubcoreMesh(core_axis_name="c", subcore_axis_name="s")

    @functools.partial(
        pl.kernel,
        mesh=mesh,
        compiler_params=pltpu.CompilerParams(use_tc_tiling_on_sc=False),
        out_type=jax.ShapeDtypeStruct((B, EMB), jnp.float32),
        scratch_types=[
            pltpu.VMEM((b_per_w,), jnp.int32),
            pltpu.VMEM((NBUF, CHUNK, EMB), jnp.float32),
            pltpu.SemaphoreType.DMA((NBUF,)),
            pltpu.SemaphoreType.DMA((NBUF,)),
        ],
    )
    def k(table_hbm, idx_hbm, out_hbm, idx_v, rows_v, gsem, ssem):
        wid = lax.axis_index("s") * NC + lax.axis_index("c")
        base = wid * b_per_w
        pltpu.sync_copy(idx_hbm.at[pl.ds(base, b_per_w)], idx_v)

        def issue_gather(c, b):
            for g in range(CHUNK // GATHER_W):
                pltpu.async_copy(
                    table_hbm.at[idx_v.at[pl.ds(c * CHUNK + g * GATHER_W, GATHER_W)]],
                    rows_v.at[b, pl.ds(g * GATHER_W, GATHER_W)],
                    gsem.at[b],
                )

        def drain_gather(b):
            pltpu.make_async_copy(
                out_hbm.at[pl.ds(0, CHUNK)], rows_v.at[b], gsem.at[b]
            ).wait()

        def drain_store(b):
            pltpu.make_async_copy(
                rows_v.at[b], out_hbm.at[pl.ds(0, CHUNK)], ssem.at[b]
            ).wait()

        for c in range(NBUF - 1):
            issue_gather(c, c % NBUF)

        @pl.loop(0, n_chunks // NBUF)
        def _(grp):
            for b in range(NBUF):
                c = grp * NBUF + b
                drain_gather(b)

                @pl.loop(0, CHUNK)
                def _(r):
                    for col in range(0, EMB, LANES):
                        rows_v.at[b, r, pl.ds(col, LANES)][...] = (
                            rows_v.at[b, r, pl.ds(col, LANES)][...] * SCALE
                        )

                pltpu.async_copy(
                    rows_v.at[b],
                    out_hbm.at[pl.ds(base + c * CHUNK, CHUNK)],
                    ssem.at[b],
                )
                c2 = c + NBUF - 1
                b2 = (b + NBUF - 1) % NBUF

                @pl.when(c2 < n_chunks)
                def _():
                    @pl.when(c2 >= NBUF)
                    def _():
                        drain_store(b2)

                    issue_gather(c2, b2)

        for b in range(NBUF):
            drain_store(b)

    return k


def kernel(x, table):
    B = x.shape[0] * x.shape[1]
    idx = x.reshape(-1).astype(jnp.int32)
    out = _emb_lookup(B)(table, idx)
    return out.reshape(x.shape[0], x.shape[1], EMB)
